# baseline (speedup 1.0000x reference)
import functools

import jax
import jax.numpy as jnp
from jax.experimental import pallas as pl


def _ffn_kernel(x_ref, w1_ref, b1_ref, w2_ref, b2_ref, o_ref):
    h = jnp.dot(
        x_ref[0].astype(jnp.bfloat16),
        w1_ref[0].astype(jnp.bfloat16),
        preferred_element_type=jnp.float32,
    )
    h = h + b1_ref[0]
    h = h * 0.5 * (1.0 + jax.lax.erf(h * (2.0 ** -0.5)))
    o_ref[0, ...] = (
        jnp.dot(
            h.astype(jnp.bfloat16),
            w2_ref[0].astype(jnp.bfloat16),
            preferred_element_type=jnp.float32,
        )
        + b2_ref[0]
    )


@functools.partial(jax.jit, static_argnames=())
def _ffn_expert0(x, w1, b1, w2, b2):
    bsz, n, d = x.shape
    ne, _, f = w1.shape
    tm = 256
    grid = (n // tm,)
    return pl.pallas_call(
        _ffn_kernel,
        grid=grid,
        in_specs=[
            pl.BlockSpec((1, tm, d), lambda i: (0, i, 0)),
            pl.BlockSpec((1, d, f), lambda i: (0, 0, 0)),
            pl.BlockSpec((1, 1, f), lambda i: (0, 0, 0)),
            pl.BlockSpec((1, f, d), lambda i: (0, 0, 0)),
            pl.BlockSpec((1, 1, d), lambda i: (0, 0, 0)),
        ],
        out_specs=pl.BlockSpec((1, tm, d), lambda i: (0, i, 0)),
        out_shape=jax.ShapeDtypeStruct((bsz, n, d), jnp.float32),
    )(x, w1, b1.reshape(ne, 1, f), w2, b2.reshape(ne, 1, d))


def kernel(x, edge_index, W_gat, att_src, att_dst, bias_gat, ln_gamma, ln_beta, W1, b1, W2, b2):
    return _ffn_expert0(x, W1, b1, W2, b2)


# speedup vs baseline: 1.0600x; 1.0600x over previous
import functools

import jax
import jax.numpy as jnp
from jax.experimental import pallas as pl


def _ffn_kernel(x_ref, w1_ref, b1_ref, w2_ref, b2_ref, o_ref):
    h = jnp.dot(
        x_ref[0].astype(jnp.bfloat16),
        w1_ref[0].astype(jnp.bfloat16),
        preferred_element_type=jnp.float32,
    )
    h = h + b1_ref[0]
    h = h * 0.5 * (1.0 + jax.lax.erf(h * (2.0 ** -0.5)))
    o_ref[0, ...] = (
        jnp.dot(
            h.astype(jnp.bfloat16),
            w2_ref[0].astype(jnp.bfloat16),
            preferred_element_type=jnp.float32,
        )
        + b2_ref[0]
    )


@functools.partial(jax.jit, static_argnames=())
def _ffn_expert0(x, w1, b1, w2, b2):
    bsz, n, d = x.shape
    ne, _, f = w1.shape
    tm = 1024
    grid = (n // tm,)
    return pl.pallas_call(
        _ffn_kernel,
        grid=grid,
        in_specs=[
            pl.BlockSpec((1, tm, d), lambda i: (0, i, 0)),
            pl.BlockSpec((1, d, f), lambda i: (0, 0, 0)),
            pl.BlockSpec((1, 1, f), lambda i: (0, 0, 0)),
            pl.BlockSpec((1, f, d), lambda i: (0, 0, 0)),
            pl.BlockSpec((1, 1, d), lambda i: (0, 0, 0)),
        ],
        out_specs=pl.BlockSpec((1, tm, d), lambda i: (0, i, 0)),
        out_shape=jax.ShapeDtypeStruct((bsz, n, d), jnp.float32),
    )(x, w1, b1.reshape(ne, 1, f), w2, b2.reshape(ne, 1, d))


def kernel(x, edge_index, W_gat, att_src, att_dst, bias_gat, ln_gamma, ln_beta, W1, b1, W2, b2):
    return _ffn_expert0(x, W1, b1, W2, b2)


# speedup vs baseline: 1.0809x; 1.0197x over previous
import functools

import jax
import jax.numpy as jnp
from jax.experimental import pallas as pl


def _ffn_kernel(x_ref, w1_ref, b1_ref, w2_ref, b2_ref, o_ref):
    h = jnp.dot(
        x_ref[0].astype(jnp.bfloat16),
        w1_ref[0].astype(jnp.bfloat16),
        preferred_element_type=jnp.float32,
    )
    h = h + b1_ref[0]
    h = h * 0.5 * (1.0 + jax.lax.erf(h * (2.0 ** -0.5)))
    o_ref[0, ...] = (
        jnp.dot(
            h.astype(jnp.bfloat16),
            w2_ref[0].astype(jnp.bfloat16),
            preferred_element_type=jnp.float32,
        )
        + b2_ref[0]
    )


@functools.partial(jax.jit, static_argnames=())
def _ffn_expert0(x, w1, b1, w2, b2):
    bsz, n, d = x.shape
    ne, _, f = w1.shape
    tm = 512
    grid = (n // tm,)
    return pl.pallas_call(
        _ffn_kernel,
        grid=grid,
        in_specs=[
            pl.BlockSpec((1, tm, d), lambda i: (0, i, 0)),
            pl.BlockSpec((1, d, f), lambda i: (0, 0, 0)),
            pl.BlockSpec((1, 1, f), lambda i: (0, 0, 0)),
            pl.BlockSpec((1, f, d), lambda i: (0, 0, 0)),
            pl.BlockSpec((1, 1, d), lambda i: (0, 0, 0)),
        ],
        out_specs=pl.BlockSpec((1, tm, d), lambda i: (0, i, 0)),
        out_shape=jax.ShapeDtypeStruct((bsz, n, d), jnp.float32),
    )(x, w1, b1.reshape(ne, 1, f), w2, b2.reshape(ne, 1, d))


def kernel(x, edge_index, W_gat, att_src, att_dst, bias_gat, ln_gamma, ln_beta, W1, b1, W2, b2):
    return _ffn_expert0(x, W1, b1, W2, b2)
